# baseline (speedup 1.0000x reference)
import functools

import jax
import jax.numpy as jnp
from jax import lax
from jax.experimental import pallas as pl
from jax.experimental.pallas import tpu as pltpu
from jax.experimental.pallas import tpu_sc as plsc

N = 10000
E = 320000
H = 128
C = 16

NC = 2
NS = 16
NW = NC * NS
EW = E // NW
EPB = 40
CPB = 50
NBLK = EW // (EPB * CPB)
RK = 80
NROWCHUNK = N // RK

_PREC = lax.Precision.HIGHEST
_BLK = 1000
_GRID = N // _BLK


def _dot(a, b):
    return jnp.dot(a, b, preferred_element_type=jnp.float32, precision=_PREC)


def _tc1_body(x_ref, linW, linb, fsW, lsW, fW, fb, l0W,
              out_ref, bg2_ref, x0_ref):
    h = jnp.maximum(_dot(x_ref[...], linW[...]) + linb[...], 0.0)
    bg = _dot(h, fsW[...])
    hs = _dot(h, lsW[...])
    out_ref[...] = jnp.maximum(bg[:, H:] * hs + bg[:, :H], 0.0)
    bg2_ref[...] = _dot(h, fW[...]) + fb[...]
    x0_ref[...] = _dot(h, l0W[...])


def _tc1(x, lin_W, lin_b, film_skip_W, lin_skip_W, film_W, film_b, lin0_W):
    full = lambda shape: pl.BlockSpec(shape, lambda i: (0,) * len(shape))
    row = lambda w: pl.BlockSpec((_BLK, w), lambda i: (i, 0))
    return pl.pallas_call(
        _tc1_body,
        grid=(_GRID,),
        in_specs=[row(H), full((H, H)), full((1, H)), full((H, 2 * H)),
                  full((H, H)), full((H, 2 * H)), full((1, 2 * H)),
                  full((H, H))],
        out_specs=[row(H), row(2 * H), row(H)],
        out_shape=[jax.ShapeDtypeStruct((N, H), jnp.float32),
                   jax.ShapeDtypeStruct((N, 2 * H), jnp.float32),
                   jax.ShapeDtypeStruct((N, H), jnp.float32)],
    )(x, lin_W, lin_b, film_skip_W, lin_skip_W, film_W, film_b, lin0_W)


ACCW = H
CNTR = 80


def _zero_rows(ref, rows, cols):
    z = jnp.zeros((16,), jnp.float32)

    def body(r, _):
        for j in range(cols // 16):
            ref[r, pl.ds(j * 16, 16)] = z
        return 0

    lax.fori_loop(0, rows, body, 0)


def _for_my_row_chunks(s, fn, rk=RK, nchunks=NROWCHUNK):
    def body(i, _):
        cid = s + i * NS

        @pl.when(cid < nchunks)
        def _():
            fn(cid * rk)

        return 0

    lax.fori_loop(0, (nchunks + NS - 1) // NS, body, 0)


def _gather_pair(tab1_hbm, idx1, buf1, tab2_hbm, idx2, buf2, sem):
    cp1 = pltpu.make_async_copy(tab1_hbm.at[idx1], buf1, sem)
    cp2 = pltpu.make_async_copy(tab2_hbm.at[idx2], buf2, sem)
    return cp1, cp2


def _sc_film(x0, bg2, src, dst):
    mesh = plsc.VectorSubcoreMesh(core_axis_name="c", subcore_axis_name="s")

    @functools.partial(
        pl.kernel,
        out_type=[jax.ShapeDtypeStruct((NC, N, ACCW), jnp.float32),
                  jax.ShapeDtypeStruct((NC, CNTR, H), jnp.float32)],
        mesh=mesh,
        scratch_types=[
            pltpu.VMEM((CPB * EPB,), jnp.int32),
            pltpu.VMEM((CPB * EPB,), jnp.int32),
            pltpu.VMEM((EPB,), jnp.int32),
            pltpu.VMEM((EPB, H), jnp.float32),
            pltpu.VMEM((EPB, H), jnp.float32),
            pltpu.VMEM((EPB, 2 * H), jnp.float32),
            pltpu.VMEM((EPB, 2 * H), jnp.float32),
            pltpu.VMEM((CNTR, H), jnp.float32),
            pltpu.VMEM((CNTR,), jnp.int32),
            pltpu.VMEM_SHARED((N, ACCW), jnp.float32),
            pltpu.VMEM_SHARED((CNTR, H), jnp.float32),
            pltpu.SemaphoreType.DMA,
            pltpu.SemaphoreType.DMA,
        ],
        compiler_params=pltpu.CompilerParams(needs_layout_passes=False),
    )
    def k(x0_hbm, bg2_hbm, src_hbm, dst_hbm, out_hbm, cnt_hbm,
          sblk, dblk, dcur, xj0, xj1, bg0, bg1, cnt, rowids, acc, cacc,
          sem0, sem1):
        c = lax.axis_index("c")
        s = lax.axis_index("s")
        w = c * NS + s
        base = w * EW

        _zero_rows(cnt, CNTR, H)

        def zrow(r, _):
            rowids[pl.ds(r * 16, 16)] = lax.iota(jnp.int32, 16) + r * 16
            return 0

        lax.fori_loop(0, CNTR // 16, zrow, 0)
        _for_my_row_chunks(
            s, lambda off: pltpu.sync_copy(cnt, acc.at[pl.ds(off, RK)]))

        @pl.when(s < CNTR // 8)
        def _():
            pltpu.sync_copy(cnt.at[pl.ds(0, 8)], cacc.at[pl.ds(s * 8, 8)])

        plsc.subcore_barrier()

        ones16 = jnp.ones((16,), jnp.float32)
        lanes = lax.iota(jnp.int32, 16)
        bufs = ((xj0, bg0, sem0), (xj1, bg1, sem1))

        def issue(lc, b):
            xj, bg, sem = bufs[b]
            cp1, cp2 = _gather_pair(x0_hbm, sblk.at[pl.ds(lc * EPB, EPB)],
                                    xj, bg2_hbm,
                                    dblk.at[pl.ds(lc * EPB, EPB)], bg, sem)
            cp1.start()
            cp2.start()

        def process(lc, b):
            xj, bg, sem = bufs[b]
            cp1, cp2 = _gather_pair(x0_hbm, sblk.at[pl.ds(lc * EPB, EPB)],
                                    xj, bg2_hbm,
                                    dblk.at[pl.ds(lc * EPB, EPB)], bg, sem)
            cp1.wait()
            cp2.wait()
            for o in (0, 16, EPB - 16):
                dcur[pl.ds(o, 16)] = dblk[pl.ds(lc * EPB + o, 16)]

            @plsc.parallel_loop(0, EPB, 1, unroll=8)
            def edge(e):
                for j in range(H // 16):
                    gv = bg[e, pl.ds(H + j * 16, 16)]
                    bv = bg[e, pl.ds(j * 16, 16)]
                    xv = xj[e, pl.ds(j * 16, 16)]
                    xj[e, pl.ds(j * 16, 16)] = jnp.maximum(gv * xv + bv, 0.0)
                jj = jnp.minimum((e >> 4) << 4, EPB - 16)
                dvec = dcur[pl.ds(jj, 16)]
                plsc.addupdate_scatter(cnt, [dvec >> 7, dvec & 127], ones16,
                                       mask=lanes == (e - jj))

            pltpu.sync_copy(xj, acc.at[dcur], add=True)

        def block(blk, _):
            boff = base + blk * CPB * EPB
            pltpu.sync_copy(src_hbm.at[pl.ds(boff, CPB * EPB)], sblk)
            pltpu.sync_copy(dst_hbm.at[pl.ds(boff, CPB * EPB)], dblk)
            issue(0, 0)

            def pair(p, _):
                lc0 = 2 * p
                issue(lc0 + 1, 1)
                process(lc0, 0)

                @pl.when(p < CPB // 2 - 1)
                def _():
                    issue(lc0 + 2, 0)

                process(lc0 + 1, 1)
                return 0

            lax.fori_loop(0, CPB // 2, pair, 0)
            return 0

        lax.fori_loop(0, NBLK, block, 0)
        pltpu.sync_copy(cnt, cacc.at[rowids], add=True)
        plsc.subcore_barrier()

        _for_my_row_chunks(
            s, lambda off: pltpu.sync_copy(acc.at[pl.ds(off, RK)],
                                           out_hbm.at[c, pl.ds(off, RK)]))

        @pl.when(s < CNTR // 8)
        def _():
            pltpu.sync_copy(cacc.at[pl.ds(s * 8, 8)],
                            cnt_hbm.at[c, pl.ds(s * 8, 8)])

    return k(x0, bg2, src, dst)


def _tc2_body(out_ref, p_ref, cnt_ref, keyW, keyb, qW, qb, vW, vb, rgW,
              k_ref, qv_ref, skip_ref):
    p = p_ref[0] + p_ref[1]
    cnt = jnp.maximum(cnt_ref[0] + cnt_ref[1], 1.0)
    h2 = jnp.maximum(out_ref[...] + p[:, :H] / cnt, 0.0)
    k = _dot(h2, keyW[...]) + keyb[...]
    q = _dot(h2, qW[...]) + qb[...]
    v = _dot(h2, vW[...]) + vb[...]
    k_ref[...] = k
    qv_ref[...] = jnp.concatenate([q, v], axis=1)
    skip_ref[...] = _dot(h2, rgW[...])


def _tc2(out, p, cnt, key_W, key_b, query_W, query_b, value_W, value_b,
         rg_skip_W):
    full = lambda shape: pl.BlockSpec(shape, lambda i: (0,) * len(shape))
    row = lambda w: pl.BlockSpec((_BLK, w), lambda i: (i, 0))
    prow = pl.BlockSpec((NC, _BLK, ACCW), lambda i: (0, i, 0))
    crow = pl.BlockSpec((NC, _BLK, 1), lambda i: (0, i, 0))
    return pl.pallas_call(
        _tc2_body,
        grid=(_GRID,),
        in_specs=[row(H), prow, crow, full((H, H)), full((1, H)),
                  full((H, H)), full((1, H)), full((H, H)), full((1, H)),
                  full((H, H))],
        out_specs=[row(H), row(2 * H), row(H)],
        out_shape=[jax.ShapeDtypeStruct((N, H), jnp.float32),
                   jax.ShapeDtypeStruct((N, 2 * H), jnp.float32),
                   jax.ShapeDtypeStruct((N, H), jnp.float32)],
    )(out, p, cnt, key_W, key_b, query_W, query_b, value_W, value_b,
      rg_skip_W)


def _sc_resgated(kk, qv, src, dst):
    mesh = plsc.VectorSubcoreMesh(core_axis_name="c", subcore_axis_name="s")

    @functools.partial(
        pl.kernel,
        out_type=jax.ShapeDtypeStruct((NC, N, H), jnp.float32),
        mesh=mesh,
        scratch_types=[
            pltpu.VMEM((CPB * EPB,), jnp.int32),
            pltpu.VMEM((CPB * EPB,), jnp.int32),
            pltpu.VMEM((EPB,), jnp.int32),
            pltpu.VMEM((EPB, H), jnp.float32),
            pltpu.VMEM((EPB, H), jnp.float32),
            pltpu.VMEM((EPB, 2 * H), jnp.float32),
            pltpu.VMEM((EPB, 2 * H), jnp.float32),
            pltpu.VMEM_SHARED((N, H), jnp.float32),
            pltpu.SemaphoreType.DMA,
            pltpu.SemaphoreType.DMA,
        ],
        compiler_params=pltpu.CompilerParams(needs_layout_passes=False),
    )
    def k(k_hbm, qv_hbm, src_hbm, dst_hbm, out_hbm,
          sblk, dblk, dcur, kd0, kd1, qv0, qv1, acc, sem0, sem1):
        c = lax.axis_index("c")
        s = lax.axis_index("s")
        w = c * NS + s
        base = w * EW

        _zero_rows(kd0, EPB, H)
        _for_my_row_chunks(
            s, lambda off: pltpu.sync_copy(kd0, acc.at[pl.ds(off, EPB)]),
            rk=EPB, nchunks=N // EPB)
        plsc.subcore_barrier()

        bufs = ((kd0, qv0, sem0), (kd1, qv1, sem1))

        def issue(lc, b):
            kd, qvb, sem = bufs[b]
            cp1, cp2 = _gather_pair(k_hbm, dblk.at[pl.ds(lc * EPB, EPB)],
                                    kd, qv_hbm,
                                    sblk.at[pl.ds(lc * EPB, EPB)], qvb, sem)
            cp1.start()
            cp2.start()

        def process(lc, b):
            kd, qvb, sem = bufs[b]
            cp1, cp2 = _gather_pair(k_hbm, dblk.at[pl.ds(lc * EPB, EPB)],
                                    kd, qv_hbm,
                                    sblk.at[pl.ds(lc * EPB, EPB)], qvb, sem)
            cp1.wait()
            cp2.wait()
            for o in (0, 16, EPB - 16):
                dcur[pl.ds(o, 16)] = dblk[pl.ds(lc * EPB + o, 16)]

            @plsc.parallel_loop(0, EPB, 1, unroll=8)
            def edge(e):
                for j in range(H // 16):
                    kv = kd[e, pl.ds(j * 16, 16)]
                    qvj = qvb[e, pl.ds(j * 16, 16)]
                    vv = qvb[e, pl.ds(H + j * 16, 16)]
                    eta = 1.0 / (1.0 + jnp.exp(-(kv + qvj)))
                    kd[e, pl.ds(j * 16, 16)] = eta * vv

            pltpu.sync_copy(kd, acc.at[dcur], add=True)

        def block(blk, _):
            boff = base + blk * CPB * EPB
            pltpu.sync_copy(src_hbm.at[pl.ds(boff, CPB * EPB)], sblk)
            pltpu.sync_copy(dst_hbm.at[pl.ds(boff, CPB * EPB)], dblk)
            issue(0, 0)

            def pair(p, _):
                lc0 = 2 * p
                issue(lc0 + 1, 1)
                process(lc0, 0)

                @pl.when(p < CPB // 2 - 1)
                def _():
                    issue(lc0 + 2, 0)

                process(lc0 + 1, 1)
                return 0

            lax.fori_loop(0, CPB // 2, pair, 0)
            return 0

        lax.fori_loop(0, NBLK, block, 0)
        plsc.subcore_barrier()

        _for_my_row_chunks(
            s, lambda off: pltpu.sync_copy(acc.at[pl.ds(off, RK)],
                                           out_hbm.at[c, pl.ds(off, RK)]))

    return k(kk, qv, src, dst)


def _tc3_body(skip_ref, p2_ref, rgb, fcW, fcb, out_ref):
    h3 = jnp.maximum(p2_ref[0] + p2_ref[1] + skip_ref[...] + rgb[...], 0.0)
    out_ref[...] = _dot(h3, fcW[...]) + fcb[...]


def _tc3(skip, p2, rg_bias, fc_W, fc_b):
    full = lambda shape: pl.BlockSpec(shape, lambda i: (0,) * len(shape))
    row = lambda w: pl.BlockSpec((_BLK, w), lambda i: (i, 0))
    prow = pl.BlockSpec((NC, _BLK, H), lambda i: (0, i, 0))
    return pl.pallas_call(
        _tc3_body,
        grid=(_GRID,),
        in_specs=[row(H), prow, full((1, H)), full((H, C)), full((1, C))],
        out_specs=row(C),
        out_shape=jax.ShapeDtypeStruct((N, C), jnp.float32),
    )(skip, p2, rg_bias, fc_W, fc_b)


def kernel(x, edge_index, lin_W, lin_b, film_skip_W, lin_skip_W, film_W,
           film_b, lin0_W, key_W, key_b, query_W, query_b, value_W, value_b,
           rg_skip_W, rg_bias, fc_W, fc_b):
    src = edge_index[0]
    dst = edge_index[1]
    out, bg2, x0 = _tc1(x, lin_W, lin_b.reshape(1, H), film_skip_W,
                        lin_skip_W, film_W, film_b.reshape(1, 2 * H), lin0_W)
    p, cnt = _sc_film(x0, bg2, src, dst)
    cnt_col = cnt.reshape(NC, CNTR * H)[:, :N].reshape(NC, N, 1)
    kk, qv, skip = _tc2(out, p, cnt_col, key_W, key_b.reshape(1, H), query_W,
                        query_b.reshape(1, H), value_W, value_b.reshape(1, H),
                        rg_skip_W)
    p2 = _sc_resgated(kk, qv, src, dst)
    logits = _tc3(skip, p2, rg_bias.reshape(1, H), fc_W, fc_b.reshape(1, C))
    return (logits, logits)


# speedup vs baseline: 2.3331x; 2.3331x over previous
import functools

import jax
import jax.numpy as jnp
from jax import lax
from jax.experimental import pallas as pl
from jax.experimental.pallas import tpu as pltpu
from jax.experimental.pallas import tpu_sc as plsc

N = 10000
E = 320000
H = 128
C = 16

NC = 2
NS = 16
NW = NC * NS
EW = E // NW
EPB = 40
CPB = 50
NBLK = EW // (EPB * CPB)
RK = 80
NROWCHUNK = N // RK

_PREC = lax.Precision.HIGHEST
_BLK = 1000
_GRID = N // _BLK


def _dot(a, b):
    return jnp.dot(a, b, preferred_element_type=jnp.float32, precision=_PREC)


def _tc1_body(x_ref, linW, linb, fsW, lsW, fW, fb, l0W,
              out_ref, bg2_ref, x0_ref):
    h = jnp.maximum(_dot(x_ref[...], linW[...]) + linb[...], 0.0)
    bg = _dot(h, fsW[...])
    hs = _dot(h, lsW[...])
    out_ref[...] = jnp.maximum(bg[:, H:] * hs + bg[:, :H], 0.0)
    bg2_ref[...] = _dot(h, fW[...]) + fb[...]
    x0_ref[...] = _dot(h, l0W[...])


def _tc1(x, lin_W, lin_b, film_skip_W, lin_skip_W, film_W, film_b, lin0_W):
    full = lambda shape: pl.BlockSpec(shape, lambda i: (0,) * len(shape))
    row = lambda w: pl.BlockSpec((_BLK, w), lambda i: (i, 0))
    return pl.pallas_call(
        _tc1_body,
        grid=(_GRID,),
        in_specs=[row(H), full((H, H)), full((1, H)), full((H, 2 * H)),
                  full((H, H)), full((H, 2 * H)), full((1, 2 * H)),
                  full((H, H))],
        out_specs=[row(H), row(2 * H), row(H)],
        out_shape=[jax.ShapeDtypeStruct((N, H), jnp.float32),
                   jax.ShapeDtypeStruct((N, 2 * H), jnp.float32),
                   jax.ShapeDtypeStruct((N, H), jnp.float32)],
    )(x, lin_W, lin_b, film_skip_W, lin_skip_W, film_W, film_b, lin0_W)


ACCW = H
CNTR = 80


def _zero_rows(ref, rows, cols):
    z = jnp.zeros((16,), jnp.float32)

    def body(r, _):
        for j in range(cols // 16):
            ref[r, pl.ds(j * 16, 16)] = z
        return 0

    lax.fori_loop(0, rows, body, 0)


def _for_my_row_chunks(s, fn, rk=RK, nchunks=NROWCHUNK):
    def body(i, _):
        cid = s + i * NS

        @pl.when(cid < nchunks)
        def _():
            fn(cid * rk)

        return 0

    lax.fori_loop(0, (nchunks + NS - 1) // NS, body, 0)


def _gather_pair(tab1_hbm, idx1, buf1, tab2_hbm, idx2, buf2, sem):
    cp1 = pltpu.make_async_copy(tab1_hbm.at[idx1], buf1, sem)
    cp2 = pltpu.make_async_copy(tab2_hbm.at[idx2], buf2, sem)
    return cp1, cp2


def _sc_film(x0, bg2, src, dst):
    mesh = plsc.VectorSubcoreMesh(core_axis_name="c", subcore_axis_name="s")

    @functools.partial(
        pl.kernel,
        out_type=[jax.ShapeDtypeStruct((NC, N, ACCW), jnp.float32),
                  jax.ShapeDtypeStruct((NC, CNTR, H), jnp.float32)],
        mesh=mesh,
        scratch_types=[
            pltpu.VMEM((CPB * EPB,), jnp.int32),
            pltpu.VMEM((CPB * EPB,), jnp.int32),
            pltpu.VMEM((EPB,), jnp.int32),
            pltpu.VMEM((EPB, H), jnp.float32),
            pltpu.VMEM((EPB, H), jnp.float32),
            pltpu.VMEM((EPB, 2 * H), jnp.float32),
            pltpu.VMEM((EPB, 2 * H), jnp.float32),
            pltpu.VMEM((CNTR, H), jnp.float32),
            pltpu.VMEM((CNTR,), jnp.int32),
            pltpu.VMEM_SHARED((N, ACCW), jnp.float32),
            pltpu.VMEM_SHARED((CNTR, H), jnp.float32),
            pltpu.SemaphoreType.DMA,
            pltpu.SemaphoreType.DMA,
        ],
        compiler_params=pltpu.CompilerParams(needs_layout_passes=False),
    )
    def k(x0_hbm, bg2_hbm, src_hbm, dst_hbm, out_hbm, cnt_hbm,
          sblk, dblk, dcur, xj0, xj1, bg0, bg1, cnt, rowids, acc, cacc,
          sem0, sem1):
        c = lax.axis_index("c")
        s = lax.axis_index("s")
        w = c * NS + s
        base = w * EW

        _zero_rows(cnt, CNTR, H)

        def zrow(r, _):
            rowids[pl.ds(r * 16, 16)] = lax.iota(jnp.int32, 16) + r * 16
            return 0

        lax.fori_loop(0, CNTR // 16, zrow, 0)
        _for_my_row_chunks(
            s, lambda off: pltpu.sync_copy(cnt, acc.at[pl.ds(off, RK)]))

        @pl.when(s < CNTR // 8)
        def _():
            pltpu.sync_copy(cnt.at[pl.ds(0, 8)], cacc.at[pl.ds(s * 8, 8)])

        plsc.subcore_barrier()

        ones16 = jnp.ones((16,), jnp.float32)
        lanes = lax.iota(jnp.int32, 16)
        bufs = ((xj0, bg0, sem0), (xj1, bg1, sem1))

        def issue(lc, b):
            xj, bg, sem = bufs[b]
            cp1, cp2 = _gather_pair(x0_hbm, sblk.at[pl.ds(lc * EPB, EPB)],
                                    xj, bg2_hbm,
                                    dblk.at[pl.ds(lc * EPB, EPB)], bg, sem)
            cp1.start()
            cp2.start()

        def process(lc, b):
            xj, bg, sem = bufs[b]
            cp1, cp2 = _gather_pair(x0_hbm, sblk.at[pl.ds(lc * EPB, EPB)],
                                    xj, bg2_hbm,
                                    dblk.at[pl.ds(lc * EPB, EPB)], bg, sem)
            cp1.wait()
            cp2.wait()
            for o in (0, 16, EPB - 16):
                dcur[pl.ds(o, 16)] = dblk[pl.ds(lc * EPB + o, 16)]

            @plsc.parallel_loop(0, EPB, 1, unroll=4)
            def edge(e):
                for j in range(H // 16):
                    gv = bg[e, pl.ds(H + j * 16, 16)]
                    bv = bg[e, pl.ds(j * 16, 16)]
                    xv = xj[e, pl.ds(j * 16, 16)]
                    xj[e, pl.ds(j * 16, 16)] = jnp.maximum(gv * xv + bv, 0.0)
                jj = jnp.minimum((e >> 4) << 4, EPB - 16)
                dvec = dcur[pl.ds(jj, 16)]
                plsc.addupdate_scatter(cnt, [dvec >> 7, dvec & 127], ones16,
                                       mask=lanes == (e - jj))

            pltpu.sync_copy(xj, acc.at[dcur], add=True)

        def block(blk, _):
            boff = base + blk * CPB * EPB
            pltpu.sync_copy(src_hbm.at[pl.ds(boff, CPB * EPB)], sblk)
            pltpu.sync_copy(dst_hbm.at[pl.ds(boff, CPB * EPB)], dblk)
            issue(0, 0)

            def pair(p, _):
                lc0 = 2 * p
                issue(lc0 + 1, 1)
                process(lc0, 0)

                @pl.when(p < CPB // 2 - 1)
                def _():
                    issue(lc0 + 2, 0)

                process(lc0 + 1, 1)
                return 0

            lax.fori_loop(0, CPB // 2, pair, 0)
            return 0

        lax.fori_loop(0, NBLK, block, 0)
        pltpu.sync_copy(cnt, cacc.at[rowids], add=True)
        plsc.subcore_barrier()

        _for_my_row_chunks(
            s, lambda off: pltpu.sync_copy(acc.at[pl.ds(off, RK)],
                                           out_hbm.at[c, pl.ds(off, RK)]))

        @pl.when(s < CNTR // 8)
        def _():
            pltpu.sync_copy(cacc.at[pl.ds(s * 8, 8)],
                            cnt_hbm.at[c, pl.ds(s * 8, 8)])

    return k(x0, bg2, src, dst)


def _tc2_body(out_ref, p_ref, cnt_ref, keyW, keyb, qW, qb, vW, vb, rgW,
              k_ref, qv_ref, skip_ref):
    p = p_ref[0] + p_ref[1]
    cnt = jnp.maximum(cnt_ref[0] + cnt_ref[1], 1.0)
    h2 = jnp.maximum(out_ref[...] + p[:, :H] / cnt, 0.0)
    k = _dot(h2, keyW[...]) + keyb[...]
    q = _dot(h2, qW[...]) + qb[...]
    v = _dot(h2, vW[...]) + vb[...]
    k_ref[...] = k
    qv_ref[...] = jnp.concatenate([q, v], axis=1)
    skip_ref[...] = _dot(h2, rgW[...])


def _tc2(out, p, cnt, key_W, key_b, query_W, query_b, value_W, value_b,
         rg_skip_W):
    full = lambda shape: pl.BlockSpec(shape, lambda i: (0,) * len(shape))
    row = lambda w: pl.BlockSpec((_BLK, w), lambda i: (i, 0))
    prow = pl.BlockSpec((NC, _BLK, ACCW), lambda i: (0, i, 0))
    crow = pl.BlockSpec((NC, _BLK, 1), lambda i: (0, i, 0))
    return pl.pallas_call(
        _tc2_body,
        grid=(_GRID,),
        in_specs=[row(H), prow, crow, full((H, H)), full((1, H)),
                  full((H, H)), full((1, H)), full((H, H)), full((1, H)),
                  full((H, H))],
        out_specs=[row(H), row(2 * H), row(H)],
        out_shape=[jax.ShapeDtypeStruct((N, H), jnp.float32),
                   jax.ShapeDtypeStruct((N, 2 * H), jnp.float32),
                   jax.ShapeDtypeStruct((N, H), jnp.float32)],
    )(out, p, cnt, key_W, key_b, query_W, query_b, value_W, value_b,
      rg_skip_W)


def _sc_resgated(kk, qv, src, dst):
    mesh = plsc.VectorSubcoreMesh(core_axis_name="c", subcore_axis_name="s")

    @functools.partial(
        pl.kernel,
        out_type=jax.ShapeDtypeStruct((NC, N, H), jnp.float32),
        mesh=mesh,
        scratch_types=[
            pltpu.VMEM((CPB * EPB,), jnp.int32),
            pltpu.VMEM((CPB * EPB,), jnp.int32),
            pltpu.VMEM((EPB,), jnp.int32),
            pltpu.VMEM((EPB, H), jnp.float32),
            pltpu.VMEM((EPB, H), jnp.float32),
            pltpu.VMEM((EPB, 2 * H), jnp.float32),
            pltpu.VMEM((EPB, 2 * H), jnp.float32),
            pltpu.VMEM_SHARED((N, H), jnp.float32),
            pltpu.SemaphoreType.DMA,
            pltpu.SemaphoreType.DMA,
        ],
        compiler_params=pltpu.CompilerParams(needs_layout_passes=False),
    )
    def k(k_hbm, qv_hbm, src_hbm, dst_hbm, out_hbm,
          sblk, dblk, dcur, kd0, kd1, qv0, qv1, acc, sem0, sem1):
        c = lax.axis_index("c")
        s = lax.axis_index("s")
        w = c * NS + s
        base = w * EW

        _zero_rows(kd0, EPB, H)
        _for_my_row_chunks(
            s, lambda off: pltpu.sync_copy(kd0, acc.at[pl.ds(off, EPB)]),
            rk=EPB, nchunks=N // EPB)
        plsc.subcore_barrier()

        bufs = ((kd0, qv0, sem0), (kd1, qv1, sem1))

        def issue(lc, b):
            kd, qvb, sem = bufs[b]
            cp1, cp2 = _gather_pair(k_hbm, dblk.at[pl.ds(lc * EPB, EPB)],
                                    kd, qv_hbm,
                                    sblk.at[pl.ds(lc * EPB, EPB)], qvb, sem)
            cp1.start()
            cp2.start()

        def process(lc, b):
            kd, qvb, sem = bufs[b]
            cp1, cp2 = _gather_pair(k_hbm, dblk.at[pl.ds(lc * EPB, EPB)],
                                    kd, qv_hbm,
                                    sblk.at[pl.ds(lc * EPB, EPB)], qvb, sem)
            cp1.wait()
            cp2.wait()
            for o in (0, 16, EPB - 16):
                dcur[pl.ds(o, 16)] = dblk[pl.ds(lc * EPB + o, 16)]

            @plsc.parallel_loop(0, EPB, 1, unroll=4)
            def edge(e):
                for j in range(H // 16):
                    kv = kd[e, pl.ds(j * 16, 16)]
                    qvj = qvb[e, pl.ds(j * 16, 16)]
                    vv = qvb[e, pl.ds(H + j * 16, 16)]
                    eta = 1.0 / (1.0 + jnp.exp(-(kv + qvj)))
                    kd[e, pl.ds(j * 16, 16)] = eta * vv

            pltpu.sync_copy(kd, acc.at[dcur], add=True)

        def block(blk, _):
            boff = base + blk * CPB * EPB
            pltpu.sync_copy(src_hbm.at[pl.ds(boff, CPB * EPB)], sblk)
            pltpu.sync_copy(dst_hbm.at[pl.ds(boff, CPB * EPB)], dblk)
            issue(0, 0)

            def pair(p, _):
                lc0 = 2 * p
                issue(lc0 + 1, 1)
                process(lc0, 0)

                @pl.when(p < CPB // 2 - 1)
                def _():
                    issue(lc0 + 2, 0)

                process(lc0 + 1, 1)
                return 0

            lax.fori_loop(0, CPB // 2, pair, 0)
            return 0

        lax.fori_loop(0, NBLK, block, 0)
        plsc.subcore_barrier()

        _for_my_row_chunks(
            s, lambda off: pltpu.sync_copy(acc.at[pl.ds(off, RK)],
                                           out_hbm.at[c, pl.ds(off, RK)]))

    return k(kk, qv, src, dst)


def _tc3_body(skip_ref, p2_ref, rgb, fcW, fcb, out_ref):
    h3 = jnp.maximum(p2_ref[0] + p2_ref[1] + skip_ref[...] + rgb[...], 0.0)
    out_ref[...] = _dot(h3, fcW[...]) + fcb[...]


def _tc3(skip, p2, rg_bias, fc_W, fc_b):
    full = lambda shape: pl.BlockSpec(shape, lambda i: (0,) * len(shape))
    row = lambda w: pl.BlockSpec((_BLK, w), lambda i: (i, 0))
    prow = pl.BlockSpec((NC, _BLK, H), lambda i: (0, i, 0))
    return pl.pallas_call(
        _tc3_body,
        grid=(_GRID,),
        in_specs=[row(H), prow, full((1, H)), full((H, C)), full((1, C))],
        out_specs=row(C),
        out_shape=jax.ShapeDtypeStruct((N, C), jnp.float32),
    )(skip, p2, rg_bias, fc_W, fc_b)


def kernel(x, edge_index, lin_W, lin_b, film_skip_W, lin_skip_W, film_W,
           film_b, lin0_W, key_W, key_b, query_W, query_b, value_W, value_b,
           rg_skip_W, rg_bias, fc_W, fc_b):
    src = edge_index[0]
    dst = edge_index[1]
    out, bg2, x0 = _tc1(x, lin_W, lin_b.reshape(1, H), film_skip_W,
                        lin_skip_W, film_W, film_b.reshape(1, 2 * H), lin0_W)
    p, cnt = _sc_film(x0, bg2, src, dst)
    cnt_col = cnt.reshape(NC, CNTR * H)[:, :N].reshape(NC, N, 1)
    kk, qv, skip = _tc2(out, p, cnt_col, key_W, key_b.reshape(1, H), query_W,
                        query_b.reshape(1, H), value_W, value_b.reshape(1, H),
                        rg_skip_W)
    p2 = _sc_resgated(kk, qv, src, dst)
    logits = _tc3(skip, p2, rg_bias.reshape(1, H), fc_W, fc_b.reshape(1, C))
    return (logits, logits)
